# baseline (speedup 1.0000x reference)
import functools

import jax
import jax.numpy as jnp
from jax import lax
from jax.experimental import pallas as pl
from jax.experimental.pallas import tpu as pltpu
from jax.experimental.pallas import tpu_sc as plsc

_EPS = 1e-5


def _fold(ln, lin):
    w = lin["w"]
    ws = ln["s"][:, None] * w
    c = jnp.sum(ws, axis=0, keepdims=True)
    b2 = (ln["o"] @ w + lin["b"])[None, :]
    return ws, c, b2


def _rs(x):
    return jnp.sum(x, axis=1, keepdims=True)


def _dot(a, b):
    return jnp.dot(a, b, preferred_element_type=jnp.float32)


def _bf(x):
    return x.astype(jnp.bfloat16)


def _tc_body(deg, d0, d2c, dec,
             ni_ref, nj_ref, ed_ref, nd_ref,
             wi0, wj0, we0, c0, b0,
             w1, c1, b1,
             w2, c2, b2,
             t1, ct, bt,
             ua, ub, cu, bu,
             w3i, w3j, c3, b3,
             w4e, w4t, c4, b4,
             nup_ref, eup_ref):
    nib = ni_ref[...]
    njb = nj_ref[...]
    ni = nib.astype(jnp.float32)
    nj = njb.astype(jnp.float32)
    ed = ed_ref[...]
    si, qi = _rs(ni), _rs(ni * ni)
    sj, qj = _rs(nj), _rs(nj * nj)
    se, qe = _rs(ed), _rs(ed * ed)

    m0 = (si + sj + se) / d0
    r0 = lax.rsqrt((qi + qj + qe) / d0 - m0 * m0 + _EPS)
    acc = _dot(nib, wi0[...]) + _dot(njb, wj0[...]) + _dot(_bf(ed), we0[...])
    h = jnp.maximum((acc - m0 * c0[...]) * r0 + b0[...], 0.0)

    dc = h.shape[-1] * 1.0
    mh = _rs(h) / dc
    rh = lax.rsqrt(_rs(h * h) / dc - mh * mh + _EPS)
    h2 = jnp.maximum((_dot(_bf(h), w1[...]) - mh * c1[...]) * rh + b1[...], 0.0)

    m2 = _rs(h2) / dc
    r2 = lax.rsqrt(_rs(h2 * h2) / dc - m2 * m2 + _EPS)
    msg = jnp.maximum(
        (_dot(_bf(h2), w2[...]) - m2 * c2[...]) * r2 + b2[...], 0.0)

    nb = msg.shape[0] // deg
    magg = jnp.sum(msg.reshape(nb, deg, msg.shape[-1]), axis=1)

    nd = nd_ref[...]
    sn, qn = _rs(nd), _rs(nd * nd)
    mn = sn / dc
    rn = lax.rsqrt(qn / dc - mn * mn + _EPS)
    n1 = jnp.maximum(
        (_dot(_bf(nd), t1[...]) - mn * ct[...]) * rn + bt[...], 0.0)

    mu = (_rs(n1) + _rs(magg)) / d2c
    ru = lax.rsqrt((_rs(n1 * n1) + _rs(magg * magg)) / d2c - mu * mu + _EPS)
    nup_ref[...] = jnp.maximum(
        (_dot(_bf(n1), ua[...]) + _dot(_bf(magg), ub[...]) - mu * cu[...]) * ru
        + bu[...], 0.0)

    m3 = (si + sj) / d2c
    r3 = lax.rsqrt((qi + qj) / d2c - m3 * m3 + _EPS)
    t = jnp.maximum(
        (_dot(nib, w3i[...]) + _dot(njb, w3j[...]) - m3 * c3[...]) * r3
        + b3[...], 0.0)

    m4 = (se + _rs(t)) / dec
    r4 = lax.rsqrt((qe + _rs(t * t)) / dec - m4 * m4 + _EPS)
    eup_ref[...] = jnp.maximum(
        (_dot(_bf(ed), w4e[...]) + _dot(_bf(t), w4t[...]) - m4 * c4[...]) * r4
        + b4[...], 0.0)


def _pick_be(e, deg):
    best = None
    for nblk in range(1, e + 1):
        if e % nblk:
            continue
        be = e // nblk
        if be % deg or (be // deg) % 8 or be % 8:
            continue
        if be <= 4096:
            best = be
            break
    assert best is not None
    return best


def _sc_gather(table, idx_i, idx_j, e):
    w = table.shape[1]
    info = plsc.get_sparse_core_info()
    nc, ns = info.num_cores, info.num_subcores
    nw = nc * ns
    chunk = 128
    perw = -(-e // (nw * chunk)) * chunk
    epad = perw * nw
    niter = perw // chunk

    pad = epad - e
    idx_i = jnp.concatenate([idx_i, jnp.zeros((pad,), jnp.int32)])
    idx_j = jnp.concatenate([idx_j, jnp.zeros((pad,), jnp.int32)])

    mesh = plsc.VectorSubcoreMesh(core_axis_name="c", subcore_axis_name="s")

    @functools.partial(
        pl.kernel,
        mesh=mesh,
        out_type=[
            jax.ShapeDtypeStruct((epad, w), jnp.int32),
            jax.ShapeDtypeStruct((epad, w), jnp.int32),
        ],
        scratch_types=[
            pltpu.VMEM((chunk,), jnp.int32),
            pltpu.VMEM((chunk, w), jnp.int32),
            pltpu.SemaphoreType.DMA,
        ],
    )
    def k(table_hbm, i_hbm, j_hbm, ni_out, nj_out, idx_v, rows_v, sem):
        wid = lax.axis_index("s") * nc + lax.axis_index("c")
        base = wid * perw

        def run(src, dst):
            def body(it, carry):
                off = base + it * chunk
                pltpu.sync_copy(src.at[pl.ds(off, chunk)], idx_v)
                pltpu.async_copy(table_hbm.at[idx_v], rows_v, sem).wait()
                pltpu.sync_copy(rows_v, dst.at[pl.ds(off, chunk)])
                return carry

            lax.fori_loop(0, niter, body, 0)

        run(i_hbm, ni_out)
        run(j_hbm, nj_out)

    return k(table, idx_i, idx_j)


def kernel(nodes, edges, i, j, params):
    n, c = nodes.shape
    e, de = edges.shape
    deg = e // n
    d0 = float(2 * c + de)
    d2c = float(2 * c)
    dec = float(de + c)

    p = params
    wi0j0e0, c0, b0 = _fold(p["msg_ln0"], p["msg_l0"])
    wi0 = wi0j0e0[:c]
    wj0 = wi0j0e0[c:2 * c]
    we0 = wi0j0e0[2 * c:]
    w1, c1, b1 = _fold(p["msg_ln1"], p["msg_l1"])
    w2, c2, b2 = _fold(p["msg_lnl"], p["msg_ll"])
    t1, ct, bt = _fold(p["ln1"], p["tr1"])
    uab, cu, bu = _fold(p["ln2"], p["up"])
    ua, ub = uab[:c], uab[c:]
    w3, c3, b3 = _fold(p["ln3"], p["tr2"])
    w3i, w3j = w3[:c], w3[c:]
    w4, c4, b4 = _fold(p["ln4"], p["eup"])
    w4e, w4t = w4[:de], w4[de:]

    packed = lax.bitcast_convert_type(
        _bf(nodes).reshape(n, c // 2, 2), jnp.int32)
    gi_pad, gj_pad = _sc_gather(packed, i.astype(jnp.int32),
                                j.astype(jnp.int32), e)
    epad = gi_pad.shape[0]
    ni_pad = lax.bitcast_convert_type(gi_pad, jnp.bfloat16).reshape(epad, c)
    nj_pad = lax.bitcast_convert_type(gj_pad, jnp.bfloat16).reshape(epad, c)

    be = _pick_be(e, deg)
    nb = be // deg
    grid = e // be

    def eb(shape):
        return pl.BlockSpec(shape, lambda b: (b, 0))

    def wb(arr):
        return pl.BlockSpec(arr.shape, lambda b: (0, 0))

    wi0, wj0, we0, w1, w2, t1, ua, ub, w3i, w3j, w4e, w4t = [
        _bf(a) for a in (wi0, wj0, we0, w1, w2, t1, ua, ub,
                         w3i, w3j, w4e, w4t)]
    wargs = [wi0, wj0, we0, c0, b0, w1, c1, b1, w2, c2, b2,
             t1, ct, bt, ua, ub, cu, bu, w3i, w3j, c3, b3,
             w4e, w4t, c4, b4]

    nup, eup = pl.pallas_call(
        functools.partial(_tc_body, deg, d0, d2c, dec),
        grid=(grid,),
        in_specs=[eb((be, c)), eb((be, c)), eb((be, de)), eb((nb, c))]
        + [wb(a) for a in wargs],
        out_specs=[eb((nb, c)), eb((be, c))],
        out_shape=[
            jax.ShapeDtypeStruct((n, c), jnp.float32),
            jax.ShapeDtypeStruct((e, c), jnp.float32),
        ],
    )(ni_pad, nj_pad, edges, nodes, *wargs)

    return nup, eup
